# baseline (speedup 1.0000x reference)
import functools

import jax
import jax.numpy as jnp
from jax import lax
from jax.experimental import pallas as pl
from jax.experimental.pallas import tpu as pltpu
from jax.experimental.pallas import tpu_sc as plsc

N = 10000
NP = 10240
H = 128
E = 320000
NC = 2
NS = 16
NW = NC * NS
CH = 128
NCH = 80
GRP = 40
HH = 64
E_PAD = NW * NCH * CH
NCHUNK = E_PAD // CH
NCH0 = 160
NCH1 = 0
CORE1_BASE = NS * NCH0
NUM_GRAPHS = 64
P = 200000
PT = 6272
P_PAD = NW * PT
ROWS_PER_TILE = NP // NS

_MESH = dict(core_axis_name="c", subcore_axis_name="s", num_cores=NC,
             num_subcores=NS)


@functools.cache
def _sc_kernels():
    mesh = plsc.VectorSubcoreMesh(**_MESH)
    deg = functools.partial(
        pl.kernel,
        out_type=jax.ShapeDtypeStruct((NC, NP, H), jnp.float32),
        mesh=mesh,
        scratch_types=[
            pltpu.VMEM((NCH, CH), jnp.int32),
            pltpu.VMEM((CH, H), jnp.float32),
            pltpu.VMEM_SHARED((NP, H), jnp.float32),
        ],
    )(_sc_deg_body)
    seg = functools.partial(
        pl.kernel,
        out_type=jax.ShapeDtypeStruct((NC, NP, H), jnp.float32),
        mesh=mesh,
        scratch_types=[
            pltpu.VMEM((GRP, CH), jnp.int32),
            pltpu.VMEM((GRP, CH), jnp.int32),
            pltpu.VMEM((2, CH, H), jnp.float32),
            pltpu.VMEM_SHARED((NP, H), jnp.float32),
            pltpu.SemaphoreType.DMA,
            pltpu.SemaphoreType.DMA,
            pltpu.SemaphoreType.DMA,
            pltpu.SemaphoreType.DMA,
        ],
    )(_sc_seg_body)
    seg2 = functools.partial(
        pl.kernel,
        out_type=jax.ShapeDtypeStruct((NC, NP, H), jnp.float32),
        mesh=mesh,
        scratch_types=[
            pltpu.VMEM((GRP, CH), jnp.int32),
            pltpu.VMEM((GRP, CH), jnp.int32),
            pltpu.VMEM((2, CH, HH), jnp.float32),
            pltpu.VMEM_SHARED((NP, HH), jnp.float32),
            pltpu.VMEM_SHARED((NP, HH), jnp.float32),
            pltpu.SemaphoreType.DMA,
            pltpu.SemaphoreType.DMA,
            pltpu.SemaphoreType.DMA,
            pltpu.SemaphoreType.DMA,
        ],
    )(_sc_seg2_body)
    lp = functools.partial(
        pl.kernel,
        out_type=jax.ShapeDtypeStruct((P_PAD,), jnp.float32),
        mesh=mesh,
        compiler_params=pltpu.CompilerParams(needs_layout_passes=False),
        scratch_types=[
            pltpu.VMEM((NP,), jnp.float32),
            pltpu.VMEM((NP,), jnp.float32),
            pltpu.VMEM((PT,), jnp.int32),
            pltpu.VMEM((PT,), jnp.int32),
            pltpu.VMEM((PT,), jnp.float32),
        ],
    )(_sc_lp_body)
    return deg, seg, seg2, lp


def _sc_deg_body(dstf, zeros128, ones128, out, idx_v, ones_v, sh):
    c = lax.axis_index("c")
    s = lax.axis_index("s")
    wid = c * NS + s
    r0 = s * ROWS_PER_TILE
    pltpu.sync_copy(zeros128.at[pl.ds(r0, ROWS_PER_TILE)],
                    sh.at[pl.ds(r0, ROWS_PER_TILE)])
    pltpu.sync_copy(ones128, ones_v)
    pltpu.sync_copy(dstf.at[pl.ds(wid * NCH, NCH)], idx_v)
    plsc.subcore_barrier()

    def body(j, carry):
        pltpu.sync_copy(ones_v, sh.at[idx_v.at[j]], add=True)
        return carry

    lax.fori_loop(0, NCH, body, 0)
    plsc.subcore_barrier()
    pltpu.sync_copy(sh.at[pl.ds(r0, ROWS_PER_TILE)],
                    out.at[c, pl.ds(r0, ROWS_PER_TILE)])


def _sc_seg_body(tmp, srcf, dstf, zeros, out, src_v, dst_v, rows_v, sh,
                 gsem0, gsem1, ssem0, ssem1):
    c = lax.axis_index("c")
    s = lax.axis_index("s")
    r0 = s * ROWS_PER_TILE
    base = jnp.where(c == 0, s * NCH0, CORE1_BASE + s * NCH1)
    ngrp = jnp.where(c == 0, NCH0 // GRP, NCH1 // GRP)
    pltpu.sync_copy(zeros.at[pl.ds(r0, ROWS_PER_TILE)],
                    sh.at[pl.ds(r0, ROWS_PER_TILE)])
    plsc.subcore_barrier()

    def _gather(j, b, gsem):
        pltpu.async_copy(tmp.at[src_v.at[j]], rows_v.at[b], gsem)

    def _gwait(b, gsem):
        pltpu.make_async_copy(tmp.at[src_v.at[0]], rows_v.at[b], gsem).wait()

    def _scat(j, b, ssem):
        pltpu.async_copy(rows_v.at[b], sh.at[dst_v.at[j]], ssem, add=True)

    def _swait(b, ssem):
        pltpu.make_async_copy(rows_v.at[b], sh.at[dst_v.at[0]], ssem).wait()

    for g in range(NCH0 // GRP):
        @pl.when(g < ngrp)
        def _():
            pltpu.sync_copy(srcf.at[pl.ds(base + g * GRP, GRP)], src_v)
            pltpu.sync_copy(dstf.at[pl.ds(base + g * GRP, GRP)], dst_v)
            _gather(0, 0, gsem0)
            _gather(1, 1, gsem1)

            def body(t, carry):
                j0 = t * 2
                j1 = j0 + 1
                _gwait(0, gsem0)
                _scat(j0, 0, ssem0)
                _gwait(1, gsem1)
                _scat(j1, 1, ssem1)

                @pl.when(t < GRP // 2 - 1)
                def _():
                    _swait(0, ssem0)
                    _gather(j0 + 2, 0, gsem0)
                    _swait(1, ssem1)
                    _gather(j1 + 2, 1, gsem1)

                @pl.when(t == GRP // 2 - 1)
                def _():
                    _swait(0, ssem0)
                    _swait(1, ssem1)

                return carry

            lax.fori_loop(0, GRP // 2, body, 0)
    plsc.subcore_barrier()
    pltpu.sync_copy(sh.at[pl.ds(r0, ROWS_PER_TILE)],
                    out.at[c, pl.ds(r0, ROWS_PER_TILE)])


def _sc_seg2_body(tmp, src3, dst3, zeros, out, src_v, dst_v, rows_v,
                  sh_tmp, sh_agg, gsem0, gsem1, ssem0, ssem1):
    c = lax.axis_index("c")
    s = lax.axis_index("s")
    wid = c * NS + s
    r0 = s * ROWS_PER_TILE

    def _gather(j, b, gsem):
        pltpu.async_copy(sh_tmp.at[src_v.at[j]], rows_v.at[b], gsem)

    def _gwait(b, gsem):
        pltpu.make_async_copy(sh_tmp.at[src_v.at[0]], rows_v.at[b],
                              gsem).wait()

    def _scat(j, b, ssem):
        pltpu.async_copy(rows_v.at[b], sh_agg.at[dst_v.at[j]], ssem, add=True)

    def _swait(b, ssem):
        pltpu.make_async_copy(rows_v.at[b], sh_agg.at[dst_v.at[0]],
                              ssem).wait()

    for h in range(H // HH):
        pltpu.sync_copy(tmp.at[pl.ds(r0, ROWS_PER_TILE), pl.ds(h * HH, HH)],
                        sh_tmp.at[pl.ds(r0, ROWS_PER_TILE)])
        pltpu.sync_copy(zeros.at[pl.ds(r0, ROWS_PER_TILE), pl.ds(0, HH)],
                        sh_agg.at[pl.ds(r0, ROWS_PER_TILE)])
        plsc.subcore_barrier()
        for g in range(NCH // GRP):
            pltpu.sync_copy(src3.at[wid, pl.ds(g * GRP, GRP)], src_v)
            pltpu.sync_copy(dst3.at[wid, pl.ds(g * GRP, GRP)], dst_v)
            _gather(0, 0, gsem0)
            _gather(1, 1, gsem1)

            def body(t, carry):
                j0 = t * 2
                j1 = j0 + 1
                _gwait(0, gsem0)
                _scat(j0, 0, ssem0)
                _gwait(1, gsem1)
                _scat(j1, 1, ssem1)

                @pl.when(t < GRP // 2 - 1)
                def _():
                    _swait(0, ssem0)
                    _gather(j0 + 2, 0, gsem0)
                    _swait(1, ssem1)
                    _gather(j1 + 2, 1, gsem1)

                @pl.when(t == GRP // 2 - 1)
                def _():
                    _swait(0, ssem0)
                    _swait(1, ssem1)

                return carry

            lax.fori_loop(0, GRP // 2, body, 0)
        plsc.subcore_barrier()
        pltpu.sync_copy(sh_agg.at[pl.ds(r0, ROWS_PER_TILE)],
                        out.at[c, pl.ds(r0, ROWS_PER_TILE),
                               pl.ds(h * HH, HH)])


def _sc_lp_body(u, v, ta, tb, out, u_v, v_v, ta_v, tb_v, o_v):
    c = lax.axis_index("c")
    s = lax.axis_index("s")
    wid = c * NS + s
    base = wid * PT
    pltpu.sync_copy(u, u_v)
    pltpu.sync_copy(v, v_v)
    pltpu.sync_copy(ta.at[pl.ds(base, PT)], ta_v)
    pltpu.sync_copy(tb.at[pl.ds(base, PT)], tb_v)

    def body(i, carry):
        ia = ta_v[pl.ds(i * 16, 16)]
        ib = tb_v[pl.ds(i * 16, 16)]
        ga = plsc.load_gather(u_v, [ia])
        gb = plsc.load_gather(v_v, [ib])
        t = ga + gb
        p = 1.0 / (1.0 + jnp.exp(-t))
        p = jnp.minimum(jnp.maximum(p, 1e-8), 1.0 - 1e-8)
        o_v[pl.ds(i * 16, 16)] = p
        return carry

    lax.fori_loop(0, PT // 16, body, 0)
    pltpu.sync_copy(o_v, out.at[pl.ds(base, PT)])


_BLK = 1024
_GRID = NP // _BLK

_row_spec = pl.BlockSpec((_BLK, H), lambda i: (i, 0))
_w_spec = pl.BlockSpec((H, H), lambda i: (0, 0))
_b_spec = pl.BlockSpec((1, H), lambda i: (0, 0))
_parts_spec = pl.BlockSpec((NC, _BLK, H), lambda i: (0, i, 0))


def _stage_a_body(x_ref, wi_ref, bi_ref, wf_ref, bf_ref, h0_ref, tmp0_ref):
    h0 = jnp.dot(x_ref[...], wi_ref[...],
                 preferred_element_type=jnp.float32) + bi_ref[...]
    h0_ref[...] = h0
    tmp0_ref[...] = jnp.dot(h0, wf_ref[...],
                            preferred_element_type=jnp.float32) + bf_ref[...]


def _stage_a(x, W_in, b_in, Wf, bf):
    return pl.pallas_call(
        _stage_a_body,
        grid=(_GRID,),
        in_specs=[_row_spec, _w_spec, _b_spec, _w_spec, _b_spec],
        out_specs=[_row_spec, _row_spec],
        out_shape=[jax.ShapeDtypeStruct((NP, H), jnp.float32)] * 2,
    )(x, W_in, b_in, Wf, bf)


def _stage_bc_body(p_ref, invd_ref, tmp_ref, s_ref, ws_ref, wn_ref, bg_ref,
                   wf_ref, bf_ref, snext_ref, tnext_ref):
    mean = (p_ref[0] + p_ref[1]) * invd_ref[...]
    h = jnp.dot(tmp_ref[...], ws_ref[...], preferred_element_type=jnp.float32)
    h = h + jnp.dot(mean, wn_ref[...], preferred_element_type=jnp.float32)
    h = jnp.maximum(h + bg_ref[...], 0.0)
    snext = s_ref[...] + h
    snext_ref[...] = snext
    tnext_ref[...] = jnp.dot(snext, wf_ref[...],
                             preferred_element_type=jnp.float32) + bf_ref[...]


def _stage_bc(parts, invd, tmp, s, Ws, Wn, bg, Wf, bf):
    return pl.pallas_call(
        _stage_bc_body,
        grid=(_GRID,),
        in_specs=[_parts_spec, _row_spec, _row_spec, _row_spec,
                  _w_spec, _w_spec, _b_spec, _w_spec, _b_spec],
        out_specs=[_row_spec, _row_spec],
        out_shape=[jax.ShapeDtypeStruct((NP, H), jnp.float32)] * 2,
    )(parts, invd, tmp, s, Ws, Wn, bg, Wf, bf)


def _stage_d_body(p_ref, invd_ref, tmp_ref, s_ref, batch_ref,
                  ws_ref, wn_ref, bg_ref, wf_ref, bf_ref,
                  wg1_ref, bg1_ref, wn1_ref, bn1_ref, wn2_ref, bn2_ref,
                  wuv_ref, buv_ref, wg2_ref, bg2_ref,
                  nc_ref, uv_ref, gc_ref, pool_acc):
    i = pl.program_id(0)
    mean = (p_ref[0] + p_ref[1]) * invd_ref[...]
    h = jnp.dot(tmp_ref[...], ws_ref[...], preferred_element_type=jnp.float32)
    h = h + jnp.dot(mean, wn_ref[...], preferred_element_type=jnp.float32)
    h = jnp.maximum(h + bg_ref[...], 0.0)
    s3 = s_ref[...] + h
    out = jnp.dot(s3, wf_ref[...],
                  preferred_element_type=jnp.float32) + bf_ref[...]
    t = jnp.maximum(
        jnp.dot(out, wg1_ref[...], preferred_element_type=jnp.float32)
        + bg1_ref[...], 0.0)
    ids = batch_ref[0]
    io = lax.broadcasted_iota(jnp.int32, (NUM_GRAPHS, _BLK), 0)
    onehot = (io == ids).astype(jnp.float32)

    @pl.when(i == 0)
    def _():
        pool_acc[...] = jnp.zeros_like(pool_acc)

    pool_acc[...] += jnp.dot(onehot, t, preferred_element_type=jnp.float32)
    nc1 = jnp.maximum(
        jnp.dot(out, wn1_ref[...], preferred_element_type=jnp.float32)
        + bn1_ref[...], 0.0)
    nc_ref[...] = jnp.dot(nc1, wn2_ref[...],
                          preferred_element_type=jnp.float32) + bn2_ref[...]
    uv_ref[...] = jnp.dot(out, wuv_ref[...],
                          preferred_element_type=jnp.float32) + buv_ref[...]

    @pl.when(i == _GRID - 1)
    def _():
        gc_ref[...] = jnp.dot(pool_acc[...], wg2_ref[...],
                              preferred_element_type=jnp.float32) + bg2_ref[...]


def _stage_d(parts, invd, tmp, s, batch3, Ws, Wn, bg, Wf, bf,
             Wg1, bg1, Wn1, bn1, Wn2p, bn2p, Wuv, buv, Wg2p, bg2p):
    return pl.pallas_call(
        _stage_d_body,
        grid=(_GRID,),
        in_specs=[_parts_spec, _row_spec, _row_spec, _row_spec,
                  pl.BlockSpec((1, 1, _BLK), lambda i: (i, 0, 0)),
                  _w_spec, _w_spec, _b_spec, _w_spec, _b_spec,
                  _w_spec, _b_spec, _w_spec, _b_spec, _w_spec, _b_spec,
                  _w_spec, _b_spec, _w_spec, _b_spec],
        out_specs=[_row_spec, _row_spec,
                   pl.BlockSpec((NUM_GRAPHS, H), lambda i: (0, 0))],
        out_shape=[jax.ShapeDtypeStruct((NP, H), jnp.float32),
                   jax.ShapeDtypeStruct((NP, H), jnp.float32),
                   jax.ShapeDtypeStruct((NUM_GRAPHS, H), jnp.float32)],
        scratch_shapes=[pltpu.VMEM((NUM_GRAPHS, H), jnp.float32)],
    )(parts, invd, tmp, s, batch3, Ws, Wn, bg, Wf, bf,
      Wg1, bg1, Wn1, bn1, Wn2p, bn2p, Wuv, buv, Wg2p, bg2p)


def kernel(x, W_in, b_in, Wfuse0, bfuse0, Wfuse1, bfuse1, Wfuse2, bfuse2, Wfuse3, bfuse3, Wself0, Wneigh0, bgnn0, Wself1, Wneigh1, bgnn1, Wself2, Wneigh2, bgnn2, Wg1, bg1, Wg2, bg2, Wn1, bn1, Wn2, bn2, Wa, ba, Wlp, blp, edge_index, batch, pos_edge_index, neg_edge_index):
    f32 = jnp.float32
    x_p = jnp.pad(x, ((0, NP - N), (0, 0)))
    src_p = jnp.pad(edge_index[0], (0, E_PAD - E))
    dst_p = jnp.pad(edge_index[1], (0, E_PAD - E), constant_values=NP - 1)
    srcf = src_p.reshape(NCHUNK, CH)
    dstf = dst_p.reshape(NCHUNK, CH)
    zeros128 = jnp.zeros((NP, H), f32)
    ones128 = jnp.ones((CH, H), f32)
    batch3 = jnp.pad(batch, (0, NP - N),
                     constant_values=NUM_GRAPHS).reshape(_GRID, 1, _BLK)

    b_in2 = b_in[None, :]
    bf02, bf12, bf22, bf32 = (b[None, :] for b in (bfuse0, bfuse1, bfuse2, bfuse3))
    bg02, bg12, bg22 = (b[None, :] for b in (bgnn0, bgnn1, bgnn2))
    Wlp0 = Wlp[:H, 0]
    Wlp1 = Wlp[H:, 0]
    Wuv = jnp.zeros((H, H), f32).at[:, 0].set(Wa @ Wlp0).at[:, 1].set(Wa @ Wlp1)
    c_all = ba @ Wlp0 + ba @ Wlp1 + blp[0]
    buv = jnp.zeros((1, H), f32).at[0, 0].set(c_all)
    Wn2p = jnp.zeros((H, H), f32).at[:, :Wn2.shape[1]].set(Wn2)
    bn2p = jnp.zeros((1, H), f32).at[0, :Wn2.shape[1]].set(bn2)
    Wg2p = jnp.zeros((H, H), f32).at[:, :Wg2.shape[1]].set(Wg2)
    bg2p = jnp.zeros((1, H), f32).at[0, :Wg2.shape[1]].set(bg2)

    _sc_deg, _sc_seg, _sc_seg2, _sc_lp = _sc_kernels()
    deg_parts = _sc_deg(dstf, zeros128, ones128)
    deg = jnp.maximum(deg_parts[0, :, 0] + deg_parts[1, :, 0], 1.0)
    invd = jnp.broadcast_to((1.0 / deg)[:, None], (NP, H))

    h0, tmp0 = _stage_a(x_p, W_in, b_in2, Wfuse0, bf02)
    p0 = _sc_seg(tmp0, srcf, dstf, zeros128)
    s1, tmp1 = _stage_bc(p0, invd, tmp0, h0, Wself0, Wneigh0, bg02,
                         Wfuse1, bf12)
    p1 = _sc_seg(tmp1, srcf, dstf, zeros128)
    s2, tmp2 = _stage_bc(p1, invd, tmp1, s1, Wself1, Wneigh1, bg12,
                         Wfuse2, bf22)
    p2 = _sc_seg(tmp2, srcf, dstf, zeros128)
    nc_pad, uvm, gc_pad = _stage_d(p2, invd, tmp2, s2, batch3,
                                   Wself2, Wneigh2, bg22, Wfuse3, bf32,
                                   Wg1, bg1[None, :], Wn1, bn1[None, :],
                                   Wn2p, bn2p, Wuv, buv, Wg2p, bg2p)

    gc_out = gc_pad[:, :Wg2.shape[1]]
    nc_out = nc_pad[:N, :Wn2.shape[1]]

    u = uvm[:, 0]
    v = uvm[:, 1]
    ta = jnp.pad(jnp.concatenate([pos_edge_index[0], neg_edge_index[0]]),
                 (0, P_PAD - P))
    tb = jnp.pad(jnp.concatenate([pos_edge_index[1], neg_edge_index[1]]),
                 (0, P_PAD - P))
    lp_flat = _sc_lp(u, v, ta, tb)
    lp_out = lp_flat[:P, None]
    return (gc_out, nc_out, lp_out)


# speedup vs baseline: 3.0014x; 3.0014x over previous
import functools

import jax
import jax.numpy as jnp
from jax import lax
from jax.experimental import pallas as pl
from jax.experimental.pallas import tpu as pltpu
from jax.experimental.pallas import tpu_sc as plsc

N = 10000
NP = 10240
H = 128
E = 320000
NC = 2
NS = 16
NW = NC * NS
CH = 128
NCH = 80
GRP = 40
HH = 64
E_PAD = NW * NCH * CH
NCHUNK = E_PAD // CH
NCH0 = 80
NCH1 = 80
CORE1_BASE = NS * NCH0
NUM_GRAPHS = 64
P = 200000
PT = 6272
P_PAD = NW * PT
ROWS_PER_TILE = NP // NS

_MESH = dict(core_axis_name="c", subcore_axis_name="s", num_cores=NC,
             num_subcores=NS)


@functools.cache
def _sc_kernels():
    mesh = plsc.VectorSubcoreMesh(**_MESH)
    deg = functools.partial(
        pl.kernel,
        out_type=jax.ShapeDtypeStruct((NC, NP, H), jnp.float32),
        mesh=mesh,
        scratch_types=[
            pltpu.VMEM((NCH, CH), jnp.int32),
            pltpu.VMEM((CH, H), jnp.float32),
            pltpu.VMEM_SHARED((NP, H), jnp.float32),
        ],
    )(_sc_deg_body)
    seg = functools.partial(
        pl.kernel,
        out_type=jax.ShapeDtypeStruct((NC, NP, H), jnp.float32),
        mesh=mesh,
        scratch_types=[
            pltpu.VMEM((GRP, CH), jnp.int32),
            pltpu.VMEM((GRP, CH), jnp.int32),
            pltpu.VMEM((2, CH, H), jnp.float32),
            pltpu.VMEM_SHARED((NP, H), jnp.float32),
            pltpu.SemaphoreType.DMA,
            pltpu.SemaphoreType.DMA,
            pltpu.SemaphoreType.DMA,
            pltpu.SemaphoreType.DMA,
        ],
    )(_sc_seg_body)
    seg2 = functools.partial(
        pl.kernel,
        out_type=jax.ShapeDtypeStruct((NC, NP, H), jnp.float32),
        mesh=mesh,
        scratch_types=[
            pltpu.VMEM((GRP, CH), jnp.int32),
            pltpu.VMEM((GRP, CH), jnp.int32),
            pltpu.VMEM((2, CH, HH), jnp.float32),
            pltpu.VMEM_SHARED((NP, HH), jnp.float32),
            pltpu.VMEM_SHARED((NP, HH), jnp.float32),
            pltpu.SemaphoreType.DMA,
            pltpu.SemaphoreType.DMA,
            pltpu.SemaphoreType.DMA,
            pltpu.SemaphoreType.DMA,
        ],
    )(_sc_seg2_body)
    lp = functools.partial(
        pl.kernel,
        out_type=jax.ShapeDtypeStruct((P_PAD,), jnp.float32),
        mesh=mesh,
        compiler_params=pltpu.CompilerParams(needs_layout_passes=False),
        scratch_types=[
            pltpu.VMEM((NP,), jnp.float32),
            pltpu.VMEM((NP,), jnp.float32),
            pltpu.VMEM((PT,), jnp.int32),
            pltpu.VMEM((PT,), jnp.int32),
            pltpu.VMEM((PT,), jnp.float32),
        ],
    )(_sc_lp_body)
    return deg, seg, seg2, lp


def _sc_deg_body(dstf, zeros128, ones128, out, idx_v, ones_v, sh):
    c = lax.axis_index("c")
    s = lax.axis_index("s")
    wid = c * NS + s
    r0 = s * ROWS_PER_TILE
    pltpu.sync_copy(zeros128.at[pl.ds(r0, ROWS_PER_TILE)],
                    sh.at[pl.ds(r0, ROWS_PER_TILE)])
    pltpu.sync_copy(ones128, ones_v)
    pltpu.sync_copy(dstf.at[pl.ds(wid * NCH, NCH)], idx_v)
    plsc.subcore_barrier()

    def body(j, carry):
        pltpu.sync_copy(ones_v, sh.at[idx_v.at[j]], add=True)
        return carry

    lax.fori_loop(0, NCH, body, 0)
    plsc.subcore_barrier()
    pltpu.sync_copy(sh.at[pl.ds(r0, ROWS_PER_TILE)],
                    out.at[c, pl.ds(r0, ROWS_PER_TILE)])


def _sc_seg_body(tmp, srcf, dstf, zeros, out, src_v, dst_v, rows_v, sh,
                 gsem0, gsem1, ssem0, ssem1):
    c = lax.axis_index("c")
    s = lax.axis_index("s")
    r0 = s * ROWS_PER_TILE
    base = jnp.where(c == 0, s * NCH0, CORE1_BASE + s * NCH1)
    ngrp = jnp.where(c == 0, NCH0 // GRP, NCH1 // GRP)
    pltpu.sync_copy(zeros.at[pl.ds(r0, ROWS_PER_TILE)],
                    sh.at[pl.ds(r0, ROWS_PER_TILE)])
    plsc.subcore_barrier()

    def _gather(j, b, gsem):
        pltpu.async_copy(tmp.at[src_v.at[j]], rows_v.at[b], gsem)

    def _gwait(b, gsem):
        pltpu.make_async_copy(tmp.at[src_v.at[0]], rows_v.at[b], gsem).wait()

    def _scat(j, b, ssem):
        pltpu.async_copy(rows_v.at[b], sh.at[dst_v.at[j]], ssem, add=True)

    def _swait(b, ssem):
        pltpu.make_async_copy(rows_v.at[b], sh.at[dst_v.at[0]], ssem).wait()

    for g in range(NCH0 // GRP):
        @pl.when(g < ngrp)
        def _():
            pltpu.sync_copy(srcf.at[pl.ds(base + g * GRP, GRP)], src_v)
            pltpu.sync_copy(dstf.at[pl.ds(base + g * GRP, GRP)], dst_v)
            _gather(0, 0, gsem0)
            _gather(1, 1, gsem1)

            def body(t, carry):
                j0 = t * 2
                j1 = j0 + 1
                _gwait(0, gsem0)
                _scat(j0, 0, ssem0)
                _gwait(1, gsem1)
                _scat(j1, 1, ssem1)

                @pl.when(t < GRP // 2 - 1)
                def _():
                    _swait(0, ssem0)
                    _gather(j0 + 2, 0, gsem0)
                    _swait(1, ssem1)
                    _gather(j1 + 2, 1, gsem1)

                @pl.when(t == GRP // 2 - 1)
                def _():
                    _swait(0, ssem0)
                    _swait(1, ssem1)

                return carry

            lax.fori_loop(0, GRP // 2, body, 0)
    plsc.subcore_barrier()
    pltpu.sync_copy(sh.at[pl.ds(r0, ROWS_PER_TILE)],
                    out.at[c, pl.ds(r0, ROWS_PER_TILE)])


def _sc_seg2_body(tmp, src3, dst3, zeros, out, src_v, dst_v, rows_v,
                  sh_tmp, sh_agg, gsem0, gsem1, ssem0, ssem1):
    c = lax.axis_index("c")
    s = lax.axis_index("s")
    wid = c * NS + s
    r0 = s * ROWS_PER_TILE

    def _gather(j, b, gsem):
        pltpu.async_copy(sh_tmp.at[src_v.at[j]], rows_v.at[b], gsem)

    def _gwait(b, gsem):
        pltpu.make_async_copy(sh_tmp.at[src_v.at[0]], rows_v.at[b],
                              gsem).wait()

    def _scat(j, b, ssem):
        pltpu.async_copy(rows_v.at[b], sh_agg.at[dst_v.at[j]], ssem, add=True)

    def _swait(b, ssem):
        pltpu.make_async_copy(rows_v.at[b], sh_agg.at[dst_v.at[0]],
                              ssem).wait()

    for h in range(H // HH):
        pltpu.sync_copy(tmp.at[pl.ds(r0, ROWS_PER_TILE), pl.ds(h * HH, HH)],
                        sh_tmp.at[pl.ds(r0, ROWS_PER_TILE)])
        pltpu.sync_copy(zeros.at[pl.ds(r0, ROWS_PER_TILE), pl.ds(0, HH)],
                        sh_agg.at[pl.ds(r0, ROWS_PER_TILE)])
        plsc.subcore_barrier()
        for g in range(NCH // GRP):
            pltpu.sync_copy(src3.at[wid, pl.ds(g * GRP, GRP)], src_v)
            pltpu.sync_copy(dst3.at[wid, pl.ds(g * GRP, GRP)], dst_v)
            _gather(0, 0, gsem0)
            _gather(1, 1, gsem1)

            def body(t, carry):
                j0 = t * 2
                j1 = j0 + 1
                _gwait(0, gsem0)
                _scat(j0, 0, ssem0)
                _gwait(1, gsem1)
                _scat(j1, 1, ssem1)

                @pl.when(t < GRP // 2 - 1)
                def _():
                    _swait(0, ssem0)
                    _gather(j0 + 2, 0, gsem0)
                    _swait(1, ssem1)
                    _gather(j1 + 2, 1, gsem1)

                @pl.when(t == GRP // 2 - 1)
                def _():
                    _swait(0, ssem0)
                    _swait(1, ssem1)

                return carry

            lax.fori_loop(0, GRP // 2, body, 0)
        plsc.subcore_barrier()
        pltpu.sync_copy(sh_agg.at[pl.ds(r0, ROWS_PER_TILE)],
                        out.at[c, pl.ds(r0, ROWS_PER_TILE),
                               pl.ds(h * HH, HH)])


def _sc_lp_body(u, v, ta, tb, out, u_v, v_v, ta_v, tb_v, o_v):
    c = lax.axis_index("c")
    s = lax.axis_index("s")
    wid = c * NS + s
    base = wid * PT
    pltpu.sync_copy(u, u_v)
    pltpu.sync_copy(v, v_v)
    pltpu.sync_copy(ta.at[pl.ds(base, PT)], ta_v)
    pltpu.sync_copy(tb.at[pl.ds(base, PT)], tb_v)

    def body(i, carry):
        ia = ta_v[pl.ds(i * 16, 16)]
        ib = tb_v[pl.ds(i * 16, 16)]
        ga = plsc.load_gather(u_v, [ia])
        gb = plsc.load_gather(v_v, [ib])
        t = ga + gb
        p = 1.0 / (1.0 + jnp.exp(-t))
        p = jnp.minimum(jnp.maximum(p, 1e-8), 1.0 - 1e-8)
        o_v[pl.ds(i * 16, 16)] = p
        return carry

    lax.fori_loop(0, PT // 16, body, 0)
    pltpu.sync_copy(o_v, out.at[pl.ds(base, PT)])


_BLK = 1024
_GRID = NP // _BLK

_row_spec = pl.BlockSpec((_BLK, H), lambda i: (i, 0))
_w_spec = pl.BlockSpec((H, H), lambda i: (0, 0))
_b_spec = pl.BlockSpec((1, H), lambda i: (0, 0))
_parts_spec = pl.BlockSpec((NC, _BLK, H), lambda i: (0, i, 0))


def _stage_a_body(x_ref, wi_ref, bi_ref, wf_ref, bf_ref, h0_ref, tmp0_ref):
    h0 = jnp.dot(x_ref[...], wi_ref[...],
                 preferred_element_type=jnp.float32) + bi_ref[...]
    h0_ref[...] = h0
    tmp0_ref[...] = jnp.dot(h0, wf_ref[...],
                            preferred_element_type=jnp.float32) + bf_ref[...]


def _stage_a(x, W_in, b_in, Wf, bf):
    return pl.pallas_call(
        _stage_a_body,
        grid=(_GRID,),
        in_specs=[_row_spec, _w_spec, _b_spec, _w_spec, _b_spec],
        out_specs=[_row_spec, _row_spec],
        out_shape=[jax.ShapeDtypeStruct((NP, H), jnp.float32)] * 2,
    )(x, W_in, b_in, Wf, bf)


def _stage_bc_body(p_ref, invd_ref, tmp_ref, s_ref, ws_ref, wn_ref, bg_ref,
                   wf_ref, bf_ref, snext_ref, tnext_ref):
    mean = (p_ref[0] + p_ref[1]) * invd_ref[...]
    h = jnp.dot(tmp_ref[...], ws_ref[...], preferred_element_type=jnp.float32)
    h = h + jnp.dot(mean, wn_ref[...], preferred_element_type=jnp.float32)
    h = jnp.maximum(h + bg_ref[...], 0.0)
    snext = s_ref[...] + h
    snext_ref[...] = snext
    tnext_ref[...] = jnp.dot(snext, wf_ref[...],
                             preferred_element_type=jnp.float32) + bf_ref[...]


def _stage_bc(parts, invd, tmp, s, Ws, Wn, bg, Wf, bf):
    return pl.pallas_call(
        _stage_bc_body,
        grid=(_GRID,),
        in_specs=[_parts_spec, _row_spec, _row_spec, _row_spec,
                  _w_spec, _w_spec, _b_spec, _w_spec, _b_spec],
        out_specs=[_row_spec, _row_spec],
        out_shape=[jax.ShapeDtypeStruct((NP, H), jnp.float32)] * 2,
    )(parts, invd, tmp, s, Ws, Wn, bg, Wf, bf)


def _stage_d_body(p_ref, invd_ref, tmp_ref, s_ref, batch_ref,
                  ws_ref, wn_ref, bg_ref, wf_ref, bf_ref,
                  wg1_ref, bg1_ref, wn1_ref, bn1_ref, wn2_ref, bn2_ref,
                  wuv_ref, buv_ref, wg2_ref, bg2_ref,
                  nc_ref, uv_ref, gc_ref, pool_acc):
    i = pl.program_id(0)
    mean = (p_ref[0] + p_ref[1]) * invd_ref[...]
    h = jnp.dot(tmp_ref[...], ws_ref[...], preferred_element_type=jnp.float32)
    h = h + jnp.dot(mean, wn_ref[...], preferred_element_type=jnp.float32)
    h = jnp.maximum(h + bg_ref[...], 0.0)
    s3 = s_ref[...] + h
    out = jnp.dot(s3, wf_ref[...],
                  preferred_element_type=jnp.float32) + bf_ref[...]
    t = jnp.maximum(
        jnp.dot(out, wg1_ref[...], preferred_element_type=jnp.float32)
        + bg1_ref[...], 0.0)
    ids = batch_ref[0]
    io = lax.broadcasted_iota(jnp.int32, (NUM_GRAPHS, _BLK), 0)
    onehot = (io == ids).astype(jnp.float32)

    @pl.when(i == 0)
    def _():
        pool_acc[...] = jnp.zeros_like(pool_acc)

    pool_acc[...] += jnp.dot(onehot, t, preferred_element_type=jnp.float32)
    nc1 = jnp.maximum(
        jnp.dot(out, wn1_ref[...], preferred_element_type=jnp.float32)
        + bn1_ref[...], 0.0)
    nc_ref[...] = jnp.dot(nc1, wn2_ref[...],
                          preferred_element_type=jnp.float32) + bn2_ref[...]
    uv_ref[...] = jnp.dot(out, wuv_ref[...],
                          preferred_element_type=jnp.float32) + buv_ref[...]

    @pl.when(i == _GRID - 1)
    def _():
        gc_ref[...] = jnp.dot(pool_acc[...], wg2_ref[...],
                              preferred_element_type=jnp.float32) + bg2_ref[...]


def _stage_d(parts, invd, tmp, s, batch3, Ws, Wn, bg, Wf, bf,
             Wg1, bg1, Wn1, bn1, Wn2p, bn2p, Wuv, buv, Wg2p, bg2p):
    return pl.pallas_call(
        _stage_d_body,
        grid=(_GRID,),
        in_specs=[_parts_spec, _row_spec, _row_spec, _row_spec,
                  pl.BlockSpec((1, 1, _BLK), lambda i: (i, 0, 0)),
                  _w_spec, _w_spec, _b_spec, _w_spec, _b_spec,
                  _w_spec, _b_spec, _w_spec, _b_spec, _w_spec, _b_spec,
                  _w_spec, _b_spec, _w_spec, _b_spec],
        out_specs=[_row_spec, _row_spec,
                   pl.BlockSpec((NUM_GRAPHS, H), lambda i: (0, 0))],
        out_shape=[jax.ShapeDtypeStruct((NP, H), jnp.float32),
                   jax.ShapeDtypeStruct((NP, H), jnp.float32),
                   jax.ShapeDtypeStruct((NUM_GRAPHS, H), jnp.float32)],
        scratch_shapes=[pltpu.VMEM((NUM_GRAPHS, H), jnp.float32)],
    )(parts, invd, tmp, s, batch3, Ws, Wn, bg, Wf, bf,
      Wg1, bg1, Wn1, bn1, Wn2p, bn2p, Wuv, buv, Wg2p, bg2p)


def kernel(x, W_in, b_in, Wfuse0, bfuse0, Wfuse1, bfuse1, Wfuse2, bfuse2, Wfuse3, bfuse3, Wself0, Wneigh0, bgnn0, Wself1, Wneigh1, bgnn1, Wself2, Wneigh2, bgnn2, Wg1, bg1, Wg2, bg2, Wn1, bn1, Wn2, bn2, Wa, ba, Wlp, blp, edge_index, batch, pos_edge_index, neg_edge_index):
    f32 = jnp.float32
    x_p = jnp.pad(x, ((0, NP - N), (0, 0)))
    npad_e = E_PAD - E
    pad_src = (jnp.arange(npad_e, dtype=jnp.int32) * 97) % N
    pad_dst = N + (jnp.arange(npad_e, dtype=jnp.int32) % (NP - N))
    src_p = jnp.concatenate([edge_index[0], pad_src])
    dst_p = jnp.concatenate([edge_index[1], pad_dst])
    srcf = src_p.reshape(NCHUNK, CH)
    dstf = dst_p.reshape(NCHUNK, CH)
    zeros128 = jnp.zeros((NP, H), f32)
    ones128 = jnp.ones((CH, H), f32)
    batch3 = jnp.pad(batch, (0, NP - N),
                     constant_values=NUM_GRAPHS).reshape(_GRID, 1, _BLK)

    b_in2 = b_in[None, :]
    bf02, bf12, bf22, bf32 = (b[None, :] for b in (bfuse0, bfuse1, bfuse2, bfuse3))
    bg02, bg12, bg22 = (b[None, :] for b in (bgnn0, bgnn1, bgnn2))
    Wlp0 = Wlp[:H, 0]
    Wlp1 = Wlp[H:, 0]
    Wuv = jnp.zeros((H, H), f32).at[:, 0].set(Wa @ Wlp0).at[:, 1].set(Wa @ Wlp1)
    c_all = ba @ Wlp0 + ba @ Wlp1 + blp[0]
    buv = jnp.zeros((1, H), f32).at[0, 0].set(c_all)
    Wn2p = jnp.zeros((H, H), f32).at[:, :Wn2.shape[1]].set(Wn2)
    bn2p = jnp.zeros((1, H), f32).at[0, :Wn2.shape[1]].set(bn2)
    Wg2p = jnp.zeros((H, H), f32).at[:, :Wg2.shape[1]].set(Wg2)
    bg2p = jnp.zeros((1, H), f32).at[0, :Wg2.shape[1]].set(bg2)

    _sc_deg, _sc_seg, _sc_seg2, _sc_lp = _sc_kernels()
    deg_parts = _sc_deg(dstf, zeros128, ones128)
    deg = jnp.maximum(deg_parts[0, :, 0] + deg_parts[1, :, 0], 1.0)
    invd = jnp.broadcast_to((1.0 / deg)[:, None], (NP, H))

    h0, tmp0 = _stage_a(x_p, W_in, b_in2, Wfuse0, bf02)
    p0 = _sc_seg(tmp0, srcf, dstf, zeros128)
    s1, tmp1 = _stage_bc(p0, invd, tmp0, h0, Wself0, Wneigh0, bg02,
                         Wfuse1, bf12)
    p1 = _sc_seg(tmp1, srcf, dstf, zeros128)
    s2, tmp2 = _stage_bc(p1, invd, tmp1, s1, Wself1, Wneigh1, bg12,
                         Wfuse2, bf22)
    p2 = _sc_seg(tmp2, srcf, dstf, zeros128)
    nc_pad, uvm, gc_pad = _stage_d(p2, invd, tmp2, s2, batch3,
                                   Wself2, Wneigh2, bg22, Wfuse3, bf32,
                                   Wg1, bg1[None, :], Wn1, bn1[None, :],
                                   Wn2p, bn2p, Wuv, buv, Wg2p, bg2p)

    gc_out = gc_pad[:, :Wg2.shape[1]]
    nc_out = nc_pad[:N, :Wn2.shape[1]]

    u = uvm[:, 0]
    v = uvm[:, 1]
    ta = jnp.pad(jnp.concatenate([pos_edge_index[0], neg_edge_index[0]]),
                 (0, P_PAD - P))
    tb = jnp.pad(jnp.concatenate([pos_edge_index[1], neg_edge_index[1]]),
                 (0, P_PAD - P))
    lp_flat = _sc_lp(u, v, ta, tb)
    lp_out = lp_flat[:P, None]
    return (gc_out, nc_out, lp_out)


# speedup vs baseline: 3.0016x; 1.0000x over previous
import functools

import jax
import jax.numpy as jnp
from jax import lax
from jax.experimental import pallas as pl
from jax.experimental.pallas import tpu as pltpu
from jax.experimental.pallas import tpu_sc as plsc

N = 10000
NP = 10240
H = 128
E = 320000
NC = 2
NS = 16
NW = NC * NS
CH = 128
NCH = 80
GRP = 40
E_PAD = NW * NCH * CH
NCHUNK = E_PAD // CH
NCH0 = 80
NCH1 = 80
CORE1_BASE = NS * NCH0
NUM_GRAPHS = 64
P = 200000
PT = 6272
P_PAD = NW * PT
ROWS_PER_TILE = NP // NS

_MESH = dict(core_axis_name="c", subcore_axis_name="s", num_cores=NC,
             num_subcores=NS)


@functools.cache
def _sc_kernels():
    mesh = plsc.VectorSubcoreMesh(**_MESH)
    deg = functools.partial(
        pl.kernel,
        out_type=jax.ShapeDtypeStruct((NC, NP, H), jnp.float32),
        mesh=mesh,
        scratch_types=[
            pltpu.VMEM((NCH, CH), jnp.int32),
            pltpu.VMEM((CH, H), jnp.float32),
            pltpu.VMEM_SHARED((NP, H), jnp.float32),
        ],
    )(_sc_deg_body)
    seg = functools.partial(
        pl.kernel,
        out_type=jax.ShapeDtypeStruct((NC, NP, H), jnp.float32),
        mesh=mesh,
        scratch_types=[
            pltpu.VMEM((GRP, CH), jnp.int32),
            pltpu.VMEM((GRP, CH), jnp.int32),
            pltpu.VMEM((2, CH, H), jnp.float32),
            pltpu.VMEM_SHARED((NP, H), jnp.float32),
            pltpu.SemaphoreType.DMA,
            pltpu.SemaphoreType.DMA,
            pltpu.SemaphoreType.DMA,
            pltpu.SemaphoreType.DMA,
        ],
    )(_sc_seg_body)
    lp = functools.partial(
        pl.kernel,
        out_type=jax.ShapeDtypeStruct((P_PAD,), jnp.float32),
        mesh=mesh,
        compiler_params=pltpu.CompilerParams(needs_layout_passes=False),
        scratch_types=[
            pltpu.VMEM((NP,), jnp.float32),
            pltpu.VMEM((NP,), jnp.float32),
            pltpu.VMEM((PT,), jnp.int32),
            pltpu.VMEM((PT,), jnp.int32),
            pltpu.VMEM((PT,), jnp.float32),
        ],
    )(_sc_lp_body)
    return deg, seg, lp


def _sc_deg_body(dstf, zeros128, ones128, out, idx_v, ones_v, sh):
    c = lax.axis_index("c")
    s = lax.axis_index("s")
    wid = c * NS + s
    r0 = s * ROWS_PER_TILE
    pltpu.sync_copy(zeros128.at[pl.ds(r0, ROWS_PER_TILE)],
                    sh.at[pl.ds(r0, ROWS_PER_TILE)])
    pltpu.sync_copy(ones128, ones_v)
    pltpu.sync_copy(dstf.at[pl.ds(wid * NCH, NCH)], idx_v)
    plsc.subcore_barrier()

    def body(j, carry):
        pltpu.sync_copy(ones_v, sh.at[idx_v.at[j]], add=True)
        return carry

    lax.fori_loop(0, NCH, body, 0)
    plsc.subcore_barrier()
    pltpu.sync_copy(sh.at[pl.ds(r0, ROWS_PER_TILE)],
                    out.at[c, pl.ds(r0, ROWS_PER_TILE)])


def _sc_seg_body(tmp, srcf, dstf, zeros, out, src_v, dst_v, rows_v, sh,
                 gsem0, gsem1, ssem0, ssem1):
    c = lax.axis_index("c")
    s = lax.axis_index("s")
    r0 = s * ROWS_PER_TILE
    base = jnp.where(c == 0, s * NCH0, CORE1_BASE + s * NCH1)
    ngrp = jnp.where(c == 0, NCH0 // GRP, NCH1 // GRP)
    pltpu.sync_copy(zeros.at[pl.ds(r0, ROWS_PER_TILE)],
                    sh.at[pl.ds(r0, ROWS_PER_TILE)])
    plsc.subcore_barrier()

    def _gather(j, b, gsem):
        pltpu.async_copy(tmp.at[src_v.at[j]], rows_v.at[b], gsem)

    def _gwait(b, gsem):
        pltpu.make_async_copy(tmp.at[src_v.at[0]], rows_v.at[b], gsem).wait()

    def _scat(j, b, ssem):
        pltpu.async_copy(rows_v.at[b], sh.at[dst_v.at[j]], ssem, add=True)

    def _swait(b, ssem):
        pltpu.make_async_copy(rows_v.at[b], sh.at[dst_v.at[0]], ssem).wait()

    for g in range(NCH0 // GRP):
        @pl.when(g < ngrp)
        def _():
            pltpu.sync_copy(srcf.at[pl.ds(base + g * GRP, GRP)], src_v)
            pltpu.sync_copy(dstf.at[pl.ds(base + g * GRP, GRP)], dst_v)
            _gather(0, 0, gsem0)
            _gather(1, 1, gsem1)

            def body(t, carry):
                j0 = t * 2
                j1 = j0 + 1
                _gwait(0, gsem0)
                _scat(j0, 0, ssem0)
                _gwait(1, gsem1)
                _scat(j1, 1, ssem1)

                @pl.when(t < GRP // 2 - 1)
                def _():
                    _swait(0, ssem0)
                    _gather(j0 + 2, 0, gsem0)
                    _swait(1, ssem1)
                    _gather(j1 + 2, 1, gsem1)

                @pl.when(t == GRP // 2 - 1)
                def _():
                    _swait(0, ssem0)
                    _swait(1, ssem1)

                return carry

            lax.fori_loop(0, GRP // 2, body, 0)
    plsc.subcore_barrier()
    pltpu.sync_copy(sh.at[pl.ds(r0, ROWS_PER_TILE)],
                    out.at[c, pl.ds(r0, ROWS_PER_TILE)])


def _sc_lp_body(u, v, ta, tb, out, u_v, v_v, ta_v, tb_v, o_v):
    c = lax.axis_index("c")
    s = lax.axis_index("s")
    wid = c * NS + s
    base = wid * PT
    pltpu.sync_copy(u, u_v)
    pltpu.sync_copy(v, v_v)
    pltpu.sync_copy(ta.at[pl.ds(base, PT)], ta_v)
    pltpu.sync_copy(tb.at[pl.ds(base, PT)], tb_v)

    def body(i, carry):
        ia = ta_v[pl.ds(i * 16, 16)]
        ib = tb_v[pl.ds(i * 16, 16)]
        ga = plsc.load_gather(u_v, [ia])
        gb = plsc.load_gather(v_v, [ib])
        t = ga + gb
        p = 1.0 / (1.0 + jnp.exp(-t))
        p = jnp.minimum(jnp.maximum(p, 1e-8), 1.0 - 1e-8)
        o_v[pl.ds(i * 16, 16)] = p
        return carry

    lax.fori_loop(0, PT // 16, body, 0)
    pltpu.sync_copy(o_v, out.at[pl.ds(base, PT)])


_BLK = 1024
_GRID = NP // _BLK

_row_spec = pl.BlockSpec((_BLK, H), lambda i: (i, 0))
_w_spec = pl.BlockSpec((H, H), lambda i: (0, 0))
_b_spec = pl.BlockSpec((1, H), lambda i: (0, 0))
_parts_spec = pl.BlockSpec((NC, _BLK, H), lambda i: (0, i, 0))


def _stage_a_body(x_ref, wi_ref, bi_ref, wf_ref, bf_ref, h0_ref, tmp0_ref):
    h0 = jnp.dot(x_ref[...], wi_ref[...],
                 preferred_element_type=jnp.float32) + bi_ref[...]
    h0_ref[...] = h0
    tmp0_ref[...] = jnp.dot(h0, wf_ref[...],
                            preferred_element_type=jnp.float32) + bf_ref[...]


def _stage_a(x, W_in, b_in, Wf, bf):
    return pl.pallas_call(
        _stage_a_body,
        grid=(_GRID,),
        in_specs=[_row_spec, _w_spec, _b_spec, _w_spec, _b_spec],
        out_specs=[_row_spec, _row_spec],
        out_shape=[jax.ShapeDtypeStruct((NP, H), jnp.float32)] * 2,
    )(x, W_in, b_in, Wf, bf)


def _stage_bc_body(p_ref, invd_ref, tmp_ref, s_ref, ws_ref, wn_ref, bg_ref,
                   wf_ref, bf_ref, snext_ref, tnext_ref):
    mean = (p_ref[0] + p_ref[1]) * invd_ref[...]
    h = jnp.dot(tmp_ref[...], ws_ref[...], preferred_element_type=jnp.float32)
    h = h + jnp.dot(mean, wn_ref[...], preferred_element_type=jnp.float32)
    h = jnp.maximum(h + bg_ref[...], 0.0)
    snext = s_ref[...] + h
    snext_ref[...] = snext
    tnext_ref[...] = jnp.dot(snext, wf_ref[...],
                             preferred_element_type=jnp.float32) + bf_ref[...]


def _stage_bc(parts, invd, tmp, s, Ws, Wn, bg, Wf, bf):
    return pl.pallas_call(
        _stage_bc_body,
        grid=(_GRID,),
        in_specs=[_parts_spec, _row_spec, _row_spec, _row_spec,
                  _w_spec, _w_spec, _b_spec, _w_spec, _b_spec],
        out_specs=[_row_spec, _row_spec],
        out_shape=[jax.ShapeDtypeStruct((NP, H), jnp.float32)] * 2,
    )(parts, invd, tmp, s, Ws, Wn, bg, Wf, bf)


def _stage_d_body(p_ref, invd_ref, tmp_ref, s_ref, batch_ref,
                  ws_ref, wn_ref, bg_ref, wf_ref, bf_ref,
                  wg1_ref, bg1_ref, wn1_ref, bn1_ref, wn2_ref, bn2_ref,
                  wuv_ref, buv_ref, wg2_ref, bg2_ref,
                  nc_ref, uv_ref, gc_ref, pool_acc):
    i = pl.program_id(0)
    mean = (p_ref[0] + p_ref[1]) * invd_ref[...]
    h = jnp.dot(tmp_ref[...], ws_ref[...], preferred_element_type=jnp.float32)
    h = h + jnp.dot(mean, wn_ref[...], preferred_element_type=jnp.float32)
    h = jnp.maximum(h + bg_ref[...], 0.0)
    s3 = s_ref[...] + h
    out = jnp.dot(s3, wf_ref[...],
                  preferred_element_type=jnp.float32) + bf_ref[...]
    t = jnp.maximum(
        jnp.dot(out, wg1_ref[...], preferred_element_type=jnp.float32)
        + bg1_ref[...], 0.0)
    ids = batch_ref[0]
    io = lax.broadcasted_iota(jnp.int32, (NUM_GRAPHS, _BLK), 0)
    onehot = (io == ids).astype(jnp.float32)

    @pl.when(i == 0)
    def _():
        pool_acc[...] = jnp.zeros_like(pool_acc)

    pool_acc[...] += jnp.dot(onehot, t, preferred_element_type=jnp.float32)
    nc1 = jnp.maximum(
        jnp.dot(out, wn1_ref[...], preferred_element_type=jnp.float32)
        + bn1_ref[...], 0.0)
    nc_ref[...] = jnp.dot(nc1, wn2_ref[...],
                          preferred_element_type=jnp.float32) + bn2_ref[...]
    uv_ref[...] = jnp.dot(out, wuv_ref[...],
                          preferred_element_type=jnp.float32) + buv_ref[...]

    @pl.when(i == _GRID - 1)
    def _():
        gc_ref[...] = jnp.dot(pool_acc[...], wg2_ref[...],
                              preferred_element_type=jnp.float32) + bg2_ref[...]


def _stage_d(parts, invd, tmp, s, batch3, Ws, Wn, bg, Wf, bf,
             Wg1, bg1, Wn1, bn1, Wn2p, bn2p, Wuv, buv, Wg2p, bg2p):
    return pl.pallas_call(
        _stage_d_body,
        grid=(_GRID,),
        in_specs=[_parts_spec, _row_spec, _row_spec, _row_spec,
                  pl.BlockSpec((1, 1, _BLK), lambda i: (i, 0, 0)),
                  _w_spec, _w_spec, _b_spec, _w_spec, _b_spec,
                  _w_spec, _b_spec, _w_spec, _b_spec, _w_spec, _b_spec,
                  _w_spec, _b_spec, _w_spec, _b_spec],
        out_specs=[_row_spec, _row_spec,
                   pl.BlockSpec((NUM_GRAPHS, H), lambda i: (0, 0))],
        out_shape=[jax.ShapeDtypeStruct((NP, H), jnp.float32),
                   jax.ShapeDtypeStruct((NP, H), jnp.float32),
                   jax.ShapeDtypeStruct((NUM_GRAPHS, H), jnp.float32)],
        scratch_shapes=[pltpu.VMEM((NUM_GRAPHS, H), jnp.float32)],
    )(parts, invd, tmp, s, batch3, Ws, Wn, bg, Wf, bf,
      Wg1, bg1, Wn1, bn1, Wn2p, bn2p, Wuv, buv, Wg2p, bg2p)


def kernel(x, W_in, b_in, Wfuse0, bfuse0, Wfuse1, bfuse1, Wfuse2, bfuse2, Wfuse3, bfuse3, Wself0, Wneigh0, bgnn0, Wself1, Wneigh1, bgnn1, Wself2, Wneigh2, bgnn2, Wg1, bg1, Wg2, bg2, Wn1, bn1, Wn2, bn2, Wa, ba, Wlp, blp, edge_index, batch, pos_edge_index, neg_edge_index):
    f32 = jnp.float32
    x_p = jnp.pad(x, ((0, NP - N), (0, 0)))
    npad_e = E_PAD - E
    pad_src = (jnp.arange(npad_e, dtype=jnp.int32) * 97) % N
    pad_dst = N + (jnp.arange(npad_e, dtype=jnp.int32) % (NP - N))
    src_p = jnp.concatenate([edge_index[0], pad_src])
    dst_p = jnp.concatenate([edge_index[1], pad_dst])
    srcf = src_p.reshape(NCHUNK, CH)
    dstf = dst_p.reshape(NCHUNK, CH)
    zeros128 = jnp.zeros((NP, H), f32)
    ones128 = jnp.ones((CH, H), f32)
    batch3 = jnp.pad(batch, (0, NP - N),
                     constant_values=NUM_GRAPHS).reshape(_GRID, 1, _BLK)

    b_in2 = b_in[None, :]
    bf02, bf12, bf22, bf32 = (b[None, :] for b in (bfuse0, bfuse1, bfuse2, bfuse3))
    bg02, bg12, bg22 = (b[None, :] for b in (bgnn0, bgnn1, bgnn2))
    Wlp0 = Wlp[:H, 0]
    Wlp1 = Wlp[H:, 0]
    Wuv = jnp.zeros((H, H), f32).at[:, 0].set(Wa @ Wlp0).at[:, 1].set(Wa @ Wlp1)
    c_all = ba @ Wlp0 + ba @ Wlp1 + blp[0]
    buv = jnp.zeros((1, H), f32).at[0, 0].set(c_all)
    Wn2p = jnp.zeros((H, H), f32).at[:, :Wn2.shape[1]].set(Wn2)
    bn2p = jnp.zeros((1, H), f32).at[0, :Wn2.shape[1]].set(bn2)
    Wg2p = jnp.zeros((H, H), f32).at[:, :Wg2.shape[1]].set(Wg2)
    bg2p = jnp.zeros((1, H), f32).at[0, :Wg2.shape[1]].set(bg2)

    _sc_deg, _sc_seg, _sc_lp = _sc_kernels()
    deg_parts = _sc_deg(dstf, zeros128, ones128)
    deg = jnp.maximum(deg_parts[0, :, 0] + deg_parts[1, :, 0], 1.0)
    invd = jnp.broadcast_to((1.0 / deg)[:, None], (NP, H))

    h0, tmp0 = _stage_a(x_p, W_in, b_in2, Wfuse0, bf02)
    p0 = _sc_seg(tmp0, srcf, dstf, zeros128)
    s1, tmp1 = _stage_bc(p0, invd, tmp0, h0, Wself0, Wneigh0, bg02,
                         Wfuse1, bf12)
    p1 = _sc_seg(tmp1, srcf, dstf, zeros128)
    s2, tmp2 = _stage_bc(p1, invd, tmp1, s1, Wself1, Wneigh1, bg12,
                         Wfuse2, bf22)
    p2 = _sc_seg(tmp2, srcf, dstf, zeros128)
    nc_pad, uvm, gc_pad = _stage_d(p2, invd, tmp2, s2, batch3,
                                   Wself2, Wneigh2, bg22, Wfuse3, bf32,
                                   Wg1, bg1[None, :], Wn1, bn1[None, :],
                                   Wn2p, bn2p, Wuv, buv, Wg2p, bg2p)

    gc_out = gc_pad[:, :Wg2.shape[1]]
    nc_out = nc_pad[:N, :Wn2.shape[1]]

    u = uvm[:, 0]
    v = uvm[:, 1]
    ta = jnp.pad(jnp.concatenate([pos_edge_index[0], neg_edge_index[0]]),
                 (0, P_PAD - P))
    tb = jnp.pad(jnp.concatenate([pos_edge_index[1], neg_edge_index[1]]),
                 (0, P_PAD - P))
    lp_flat = _sc_lp(u, v, ta, tb)
    lp_out = lp_flat[:P, None]
    return (gc_out, nc_out, lp_out)
